# Initial kernel scaffold; baseline (speedup 1.0000x reference)
import jax
import jax.numpy as jnp
from jax.experimental import pallas as pl


def kernel(x, edge_index, edge_weight, batch, lin1_W, lin1_b, lin2_W, lin2_b, cls1_W, cls1_b, cls2_W, cls2_b):
    raise NotImplementedError("write your pallas kernel here")


# speedup vs baseline: 1.6352x; 1.6352x over previous
import functools

import jax
import jax.numpy as jnp
import numpy as np
from jax import lax
from jax.experimental import pallas as pl
from jax.experimental.pallas import tpu as pltpu
from jax.experimental.pallas import tpu_sc as plsc

_TS = np.array([0.25, 0.5, 0.75, 1.0], dtype=np.float64)
_DEGREE = 32
_N = 10000
_E = 320000
_DIN = 128
_HID = 128
_NOUT = 10
_NG = 64
_T = len(_TS)

_NW = 32
_FPW = _DIN // _NW
_ROW = _N * _FPW
_CH = 8000
_NCH = _E // _CH
_NPAIR = _NCH // 2
_NGRP = _CH // 16
_SLICE = _E // _NW
_PCH = 2000
_K1 = _DEGREE + 1


def _cheb_coefs():
    nint = 1000
    theta = np.linspace(0.0, np.pi, nint)
    dtheta = theta[1] - theta[0]
    lam = np.cos(theta) + 1.0
    coefs = np.zeros((_T, _K1), dtype=np.float64)
    for i, t in enumerate(_TS):
        f = np.exp(-t * lam)
        for k in range(_K1):
            y = f * np.cos(k * theta)
            a = (2.0 / np.pi) * dtheta * (0.5 * y[0] + 0.5 * y[-1] + np.sum(y[1:-1]))
            if k == 0:
                a *= 0.5
            coefs[i, k] = a
    return jnp.asarray(coefs, dtype=jnp.float32)


_COEFS = _cheb_coefs()

_sc_mesh = plsc.VectorSubcoreMesh(core_axis_name="c", subcore_axis_name="s")
_sc_params = pltpu.CompilerParams(needs_layout_passes=False)


def _worker_id():
    return lax.axis_index("s") * 2 + lax.axis_index("c")


def _unpack_src(pk):
    return (pk << 2) & jnp.int32(0x3FFFC)


def _unpack_dst(pk):
    return (pk >> 14) & jnp.int32(-4)


@functools.partial(
    pl.kernel,
    out_type=jax.ShapeDtypeStruct((_E,), jnp.float32),
    mesh=_sc_mesh,
    compiler_params=_sc_params,
    scratch_types=[
        pltpu.VMEM((_N,), jnp.float32),
        pltpu.VMEM((_PCH,), jnp.int32),
        pltpu.VMEM((_PCH,), jnp.float32),
        pltpu.VMEM((_PCH,), jnp.float32),
    ],
)
def _norm_kernel(pk_hbm, w_hbm, nm_hbm, deg, pkb, wb, nb):
    w = _worker_id()
    zero16 = jnp.zeros((16,), jnp.float32)

    def z_body(i, _):
        deg[pl.ds(i * 16, 16)] = zero16
        return 0
    lax.fori_loop(0, _N // 16, z_body, 0)

    def deg_chunk(ci, _):
        pltpu.sync_copy(pk_hbm.at[pl.ds(ci * _PCH, _PCH)], pkb)
        pltpu.sync_copy(w_hbm.at[pl.ds(ci * _PCH, _PCH)], wb)

        def grp(g, _):
            pk = pkb[pl.ds(g * 16, 16)]
            ww = wb[pl.ds(g * 16, 16)]
            plsc.addupdate_scatter(deg, [pk >> 16], ww)
            return 0
        lax.fori_loop(0, _PCH // 16, grp, 0)
        return 0
    lax.fori_loop(0, _E // _PCH, deg_chunk, 0)

    def rs_body(i, _):
        d = deg[pl.ds(i * 16, 16)]
        bits = plsc.bitcast(d, jnp.int32)
        y = plsc.bitcast(jnp.int32(0x5F3759DF) - (bits >> 1), jnp.float32)
        for _i in range(3):
            y = y * (1.5 - 0.5 * d * y * y)
        deg[pl.ds(i * 16, 16)] = jnp.where(d > 0.0, y, 0.0)
        return 0
    lax.fori_loop(0, _N // 16, rs_body, 0)

    base = w * _SLICE

    def nm_chunk(ci, _):
        off = base + ci * _PCH
        pltpu.sync_copy(pk_hbm.at[pl.ds(off, _PCH)], pkb)
        pltpu.sync_copy(w_hbm.at[pl.ds(off, _PCH)], wb)

        def grp(g, _):
            pk = pkb[pl.ds(g * 16, 16)]
            ww = wb[pl.ds(g * 16, 16)]
            gs = plsc.load_gather(deg, [pk & jnp.int32(0xFFFF)])
            gd = plsc.load_gather(deg, [pk >> 16])
            nb[pl.ds(g * 16, 16)] = gs * ww * gd
            return 0
        lax.fori_loop(0, _PCH // 16, grp, 0)
        pltpu.sync_copy(nb, nm_hbm.at[pl.ds(off, _PCH)])
        return 0
    lax.fori_loop(0, _SLICE // _PCH, nm_chunk, 0)


@functools.partial(
    pl.kernel,
    out_type=jax.ShapeDtypeStruct((_K1, _NW, _ROW), jnp.float32),
    mesh=_sc_mesh,
    compiler_params=_sc_params,
    scratch_types=[
        pltpu.VMEM((_ROW,), jnp.float32),
        pltpu.VMEM((_ROW,), jnp.float32),
        pltpu.VMEM((_CH,), jnp.int32),
        pltpu.VMEM((_CH,), jnp.int32),
        pltpu.VMEM((_CH,), jnp.float32),
        pltpu.VMEM((_CH,), jnp.float32),
        pltpu.SemaphoreType.DMA,
        pltpu.SemaphoreType.DMA,
        pltpu.SemaphoreType.DMA,
        pltpu.SemaphoreType.DMA,
    ],
)
def _layer_kernel(xt_hbm, pk_hbm, nm_hbm, t_hbm, bufa, bufb, pk0, pk1,
                  nm0, nm1, sp0, sp1, sn0, sn1):
    w = _worker_id()
    zero16 = jnp.zeros((16,), jnp.float32)

    def start(ci, pkb, nmb, sp, sn):
        pltpu.make_async_copy(pk_hbm.at[pl.ds(ci * _CH, _CH)], pkb, sp).start()
        pltpu.make_async_copy(nm_hbm.at[pl.ds(ci * _CH, _CH)], nmb, sn).start()

    def wait(pkb, nmb, sp, sn):
        pltpu.make_async_copy(pk_hbm.at[pl.ds(0, _CH)], pkb, sp).wait()
        pltpu.make_async_copy(nm_hbm.at[pl.ds(0, _CH)], nmb, sn).wait()

    def process(pkb, nmb, srcbuf, accbuf, factor):
        def grp(g, _):
            pk = pkb[pl.ds(g * 16, 16)]
            nm = nmb[pl.ds(g * 16, 16)]
            sb = _unpack_src(pk)
            db = _unpack_dst(pk)
            nf = nm * factor
            for f in range(_FPW):
                gf = plsc.load_gather(srcbuf, [sb + f])
                plsc.addupdate_scatter(accbuf, [db + f], gf * nf)
            return 0
        lax.fori_loop(0, _NGRP, grp, 0)

    def edge_pass(srcbuf, accbuf, factor):
        start(0, pk0, nm0, sp0, sn0)
        start(1, pk1, nm1, sp1, sn1)

        def pair(p, _):
            wait(pk0, nm0, sp0, sn0)
            process(pk0, nm0, srcbuf, accbuf, factor)

            @pl.when(p < _NPAIR - 1)
            def _():
                start(2 * p + 2, pk0, nm0, sp0, sn0)
            wait(pk1, nm1, sp1, sn1)
            process(pk1, nm1, srcbuf, accbuf, factor)

            @pl.when(p < _NPAIR - 1)
            def _():
                start(2 * p + 3, pk1, nm1, sp1, sn1)
            return 0
        lax.fori_loop(0, _NPAIR, pair, 0)

    def negate(buf):
        def it(i, _):
            buf[pl.ds(i * 16, 16)] = -buf[pl.ds(i * 16, 16)]
            return 0
        lax.fori_loop(0, _ROW // 16, it, 0)

    pltpu.sync_copy(xt_hbm.at[w], bufb)
    pltpu.sync_copy(bufb, t_hbm.at[0, w])

    def z_body(i, _):
        bufa[pl.ds(i * 16, 16)] = zero16
        return 0
    lax.fori_loop(0, _ROW // 16, z_body, 0)

    def kpair(kk, _):
        k1 = 2 * kk + 1

        @pl.when(k1 > 1)
        def _():
            negate(bufa)
        f1 = jnp.where(k1 == 1, jnp.float32(-1.0), jnp.float32(-2.0))
        edge_pass(bufb, bufa, f1)
        pltpu.sync_copy(bufa, t_hbm.at[k1, w])
        negate(bufb)
        edge_pass(bufa, bufb, jnp.float32(-2.0))
        pltpu.sync_copy(bufb, t_hbm.at[k1 + 1, w])
        return 0
    lax.fori_loop(0, _K1 // 2, kpair, 0)


def _combine_body(coef_ref, t_ref, o_ref):
    o_ref[...] = jnp.maximum(
        lax.dot_general(coef_ref[...], t_ref[...], (((1,), (0,)), ((), ())),
                        preferred_element_type=jnp.float32), 0.0)


_CBLK = 12800


def _combine(tstack):
    tflat = tstack.reshape(_K1, _NW * _ROW)
    ncols = _NW * _ROW
    return pl.pallas_call(
        _combine_body,
        grid=(ncols // _CBLK,),
        in_specs=[
            pl.BlockSpec((_T, _K1), lambda i: (0, 0)),
            pl.BlockSpec((_K1, _CBLK), lambda i: (0, i)),
        ],
        out_specs=pl.BlockSpec((_T, _CBLK), lambda i: (0, i)),
        out_shape=jax.ShapeDtypeStruct((_T, ncols), jnp.float32),
    )(_COEFS, tflat)


def _lin_body(h_ref, w_ref, b_ref, o_ref):
    o_ref[...] = jnp.maximum(
        lax.dot_general(h_ref[...], w_ref[...], (((1,), (1,)), ((), ())),
                        preferred_element_type=jnp.float32) + b_ref[...], 0.0)


_LBLK = 2000


def _lin_relu(h, w, b):
    return pl.pallas_call(
        _lin_body,
        grid=(_N // _LBLK,),
        in_specs=[
            pl.BlockSpec((_LBLK, _T * _DIN), lambda i: (i, 0)),
            pl.BlockSpec(w.shape, lambda i: (0, 0)),
            pl.BlockSpec((1, w.shape[0]), lambda i: (0, 0)),
        ],
        out_specs=pl.BlockSpec((_LBLK, w.shape[0]), lambda i: (i, 0)),
        out_shape=jax.ShapeDtypeStruct((_N, w.shape[0]), jnp.float32),
    )(h, w, b.reshape(1, -1))


def _pool_body(b_ref, h_ref, s_ref, c_ref):
    i = pl.program_id(0)
    bb = b_ref[...].reshape(1, _LBLK)
    ids = lax.broadcasted_iota(jnp.int32, (_NG, _LBLK), 0)
    m = (ids == bb).astype(jnp.float32)
    ps = lax.dot_general(m, h_ref[...], (((1,), (0,)), ((), ())),
                         preferred_element_type=jnp.float32)
    pc = jnp.broadcast_to(jnp.sum(m, axis=1, keepdims=True), (_NG, 128))

    @pl.when(i == 0)
    def _():
        s_ref[...] = jnp.zeros_like(s_ref)
        c_ref[...] = jnp.zeros_like(c_ref)
    s_ref[...] += ps
    c_ref[...] += pc


def _pool(h, batch):
    b3 = batch.reshape(_N // _LBLK, 1, _LBLK)
    return pl.pallas_call(
        _pool_body,
        grid=(_N // _LBLK,),
        in_specs=[
            pl.BlockSpec((1, 1, _LBLK), lambda i: (i, 0, 0)),
            pl.BlockSpec((_LBLK, _T * _HID), lambda i: (i, 0)),
        ],
        out_specs=[
            pl.BlockSpec((_NG, _T * _HID), lambda i: (0, 0)),
            pl.BlockSpec((_NG, 128), lambda i: (0, 0)),
        ],
        out_shape=[
            jax.ShapeDtypeStruct((_NG, _T * _HID), jnp.float32),
            jax.ShapeDtypeStruct((_NG, 128), jnp.float32),
        ],
    )(b3, h)


def _head_body(s_ref, c_ref, w2_ref, b2_ref, w3_ref, b3_ref, w4_ref, b4_ref,
               o_ref):
    cnt = jnp.maximum(c_ref[:, 0:1], 1.0)
    g = s_ref[...] / cnt
    g = lax.dot_general(g, w2_ref[...], (((1,), (1,)), ((), ())),
                        preferred_element_type=jnp.float32) + b2_ref[...]
    g = jnp.maximum(g, 0.0)
    g = lax.dot_general(g, w3_ref[...], (((1,), (1,)), ((), ())),
                        preferred_element_type=jnp.float32) + b3_ref[...]
    g = jnp.maximum(g, 0.0)
    g = lax.dot_general(g, w4_ref[...], (((1,), (1,)), ((), ())),
                        preferred_element_type=jnp.float32) + b4_ref[...]
    mx = jnp.max(g, axis=1, keepdims=True)
    lse = mx + jnp.log(jnp.sum(jnp.exp(g - mx), axis=1, keepdims=True))
    o_ref[...] = g - lse


def _head(sums, cnts, lin2_W, lin2_b, cls1_W, cls1_b, cls2_W, cls2_b):
    args = (sums, cnts, lin2_W, lin2_b.reshape(1, -1), cls1_W,
            cls1_b.reshape(1, -1), cls2_W, cls2_b.reshape(1, -1))
    return pl.pallas_call(
        _head_body,
        out_shape=jax.ShapeDtypeStruct((_NG, _NOUT), jnp.float32),
    )(*args)


def _to_tiles(h):
    return h.reshape(_N, _NW, _FPW).transpose(1, 0, 2).reshape(_NW, _ROW)


def _from_combine(c):
    return c.reshape(_T, _NW, _N, _FPW).transpose(2, 0, 1, 3).reshape(_N, -1)


def kernel(x, edge_index, edge_weight, batch, lin1_W, lin1_b, lin2_W, lin2_b,
           cls1_W, cls1_b, cls2_W, cls2_b):
    src = edge_index[0].astype(jnp.int32)
    dst = edge_index[1].astype(jnp.int32)
    packed = src | (dst << 16)
    norm = _norm_kernel(packed, edge_weight)

    t1 = _layer_kernel(_to_tiles(x), packed, norm)
    h = _lin_relu(_from_combine(_combine(t1)), lin1_W, lin1_b)

    t2 = _layer_kernel(_to_tiles(h), packed, norm)
    h2 = _from_combine(_combine(t2))

    sums, cnts = _pool(h2, batch)
    return _head(sums, cnts, lin2_W, lin2_b, cls1_W, cls1_b, cls2_W, cls2_b)


# speedup vs baseline: 3.3569x; 2.0529x over previous
import functools

import jax
import jax.numpy as jnp
import numpy as np
from jax import lax
from jax.experimental import pallas as pl
from jax.experimental.pallas import tpu as pltpu
from jax.experimental.pallas import tpu_sc as plsc

_TS = np.array([0.25, 0.5, 0.75, 1.0], dtype=np.float64)
_DEGREE = 32
_N = 10000
_E = 320000
_DIN = 128
_HID = 128
_NOUT = 10
_NG = 64
_T = len(_TS)

_NW = 32
_FPW = _DIN // _NW
_ROW = _N * _FPW
_CH = 8000
_NCH = _E // _CH
_NPAIR = _NCH // 2
_NGRP = _CH // 16
_SLICE = _E // _NW
_PCH = 2000
_K1 = _DEGREE + 1


def _cheb_coefs():
    nint = 1000
    theta = np.linspace(0.0, np.pi, nint)
    dtheta = theta[1] - theta[0]
    lam = np.cos(theta) + 1.0
    coefs = np.zeros((_T, _K1), dtype=np.float64)
    for i, t in enumerate(_TS):
        f = np.exp(-t * lam)
        for k in range(_K1):
            y = f * np.cos(k * theta)
            a = (2.0 / np.pi) * dtheta * (0.5 * y[0] + 0.5 * y[-1] + np.sum(y[1:-1]))
            if k == 0:
                a *= 0.5
            coefs[i, k] = a
    return jnp.asarray(coefs, dtype=jnp.float32)


_COEFS = _cheb_coefs()

_sc_mesh = plsc.VectorSubcoreMesh(core_axis_name="c", subcore_axis_name="s")
_sc_params = pltpu.CompilerParams(needs_layout_passes=False)


def _worker_id():
    return lax.axis_index("s") * 2 + lax.axis_index("c")


def _unpack_src(pk):
    return (pk << 2) & jnp.int32(0x3FFFC)


def _unpack_dst(pk):
    return (pk >> 14) & jnp.int32(-4)


@functools.partial(
    pl.kernel,
    out_type=jax.ShapeDtypeStruct((_E,), jnp.float32),
    mesh=_sc_mesh,
    compiler_params=_sc_params,
    scratch_types=[
        pltpu.VMEM((_N,), jnp.float32),
        pltpu.VMEM((_PCH,), jnp.int32),
        pltpu.VMEM((_PCH,), jnp.float32),
        pltpu.VMEM((_PCH,), jnp.float32),
    ],
)
def _norm_kernel(pk_hbm, w_hbm, nm_hbm, deg, pkb, wb, nb):
    w = _worker_id()
    zero16 = jnp.zeros((16,), jnp.float32)

    def z_body(i, _):
        deg[pl.ds(i * 16, 16)] = zero16
        return 0
    lax.fori_loop(0, _N // 16, z_body, 0)

    def deg_chunk(ci, _):
        pltpu.sync_copy(pk_hbm.at[pl.ds(ci * _PCH, _PCH)], pkb)
        pltpu.sync_copy(w_hbm.at[pl.ds(ci * _PCH, _PCH)], wb)

        def grp(g, _):
            pk = pkb[pl.ds(g * 16, 16)]
            ww = wb[pl.ds(g * 16, 16)]
            plsc.addupdate_scatter(deg, [pk >> 16], ww)
            return 0
        lax.fori_loop(0, _PCH // 16, grp, 0)
        return 0
    lax.fori_loop(0, _E // _PCH, deg_chunk, 0)

    def rs_body(i, _):
        d = deg[pl.ds(i * 16, 16)]
        bits = plsc.bitcast(d, jnp.int32)
        y = plsc.bitcast(jnp.int32(0x5F3759DF) - (bits >> 1), jnp.float32)
        for _i in range(3):
            y = y * (1.5 - 0.5 * d * y * y)
        deg[pl.ds(i * 16, 16)] = jnp.where(d > 0.0, y, 0.0)
        return 0
    lax.fori_loop(0, _N // 16, rs_body, 0)

    base = w * _SLICE

    def nm_chunk(ci, _):
        off = base + ci * _PCH
        pltpu.sync_copy(pk_hbm.at[pl.ds(off, _PCH)], pkb)
        pltpu.sync_copy(w_hbm.at[pl.ds(off, _PCH)], wb)

        def grp(g, _):
            pk = pkb[pl.ds(g * 16, 16)]
            ww = wb[pl.ds(g * 16, 16)]
            gs = plsc.load_gather(deg, [pk & jnp.int32(0xFFFF)])
            gd = plsc.load_gather(deg, [pk >> 16])
            nb[pl.ds(g * 16, 16)] = gs * ww * gd
            return 0
        lax.fori_loop(0, _PCH // 16, grp, 0)
        pltpu.sync_copy(nb, nm_hbm.at[pl.ds(off, _PCH)])
        return 0
    lax.fori_loop(0, _SLICE // _PCH, nm_chunk, 0)


@functools.partial(
    pl.kernel,
    out_type=jax.ShapeDtypeStruct((_K1, _NW, _ROW), jnp.float32),
    mesh=_sc_mesh,
    compiler_params=_sc_params,
    scratch_types=[
        pltpu.VMEM((_ROW,), jnp.float32),
        pltpu.VMEM((_ROW,), jnp.float32),
        pltpu.VMEM((_CH,), jnp.int32),
        pltpu.VMEM((_CH,), jnp.int32),
        pltpu.VMEM((_CH,), jnp.float32),
        pltpu.VMEM((_CH,), jnp.float32),
        pltpu.SemaphoreType.DMA,
        pltpu.SemaphoreType.DMA,
        pltpu.SemaphoreType.DMA,
        pltpu.SemaphoreType.DMA,
    ],
)
def _layer_kernel(xt_hbm, pk_hbm, nm_hbm, t_hbm, bufa, bufb, pk0, pk1,
                  nm0, nm1, sp0, sp1, sn0, sn1):
    w = _worker_id()
    zero16 = jnp.zeros((16,), jnp.float32)

    def start(ci, pkb, nmb, sp, sn):
        pltpu.make_async_copy(pk_hbm.at[pl.ds(ci * _CH, _CH)], pkb, sp).start()
        pltpu.make_async_copy(nm_hbm.at[pl.ds(ci * _CH, _CH)], nmb, sn).start()

    def wait(pkb, nmb, sp, sn):
        pltpu.make_async_copy(pk_hbm.at[pl.ds(0, _CH)], pkb, sp).wait()
        pltpu.make_async_copy(nm_hbm.at[pl.ds(0, _CH)], nmb, sn).wait()

    def process(pkb, nmb, srcbuf, accbuf, factor):
        @plsc.parallel_loop(0, _NGRP, unroll=8)
        def grp(g):
            pk = pkb[pl.ds(g * 16, 16)]
            nm = nmb[pl.ds(g * 16, 16)]
            sb = _unpack_src(pk)
            db = _unpack_dst(pk)
            nf = nm * factor
            for f in range(_FPW):
                gf = plsc.load_gather(srcbuf, [sb + f])
                plsc.addupdate_scatter(accbuf, [db + f], gf * nf)

    def edge_pass(srcbuf, accbuf, factor):
        start(0, pk0, nm0, sp0, sn0)
        start(1, pk1, nm1, sp1, sn1)

        def pair(p, _):
            wait(pk0, nm0, sp0, sn0)
            process(pk0, nm0, srcbuf, accbuf, factor)

            @pl.when(p < _NPAIR - 1)
            def _():
                start(2 * p + 2, pk0, nm0, sp0, sn0)
            wait(pk1, nm1, sp1, sn1)
            process(pk1, nm1, srcbuf, accbuf, factor)

            @pl.when(p < _NPAIR - 1)
            def _():
                start(2 * p + 3, pk1, nm1, sp1, sn1)
            return 0
        lax.fori_loop(0, _NPAIR, pair, 0)

    def negate(buf):
        @plsc.parallel_loop(0, _ROW // 16, unroll=8)
        def it(i):
            buf[pl.ds(i * 16, 16)] = -buf[pl.ds(i * 16, 16)]

    pltpu.sync_copy(xt_hbm.at[w], bufb)
    pltpu.sync_copy(bufb, t_hbm.at[0, w])

    @plsc.parallel_loop(0, _ROW // 16, unroll=8)
    def z_body(i):
        bufa[pl.ds(i * 16, 16)] = zero16

    def kpair(kk, _):
        k1 = 2 * kk + 1

        @pl.when(k1 > 1)
        def _():
            negate(bufa)
        f1 = jnp.where(k1 == 1, jnp.float32(-1.0), jnp.float32(-2.0))
        edge_pass(bufb, bufa, f1)
        pltpu.sync_copy(bufa, t_hbm.at[k1, w])
        negate(bufb)
        edge_pass(bufa, bufb, jnp.float32(-2.0))
        pltpu.sync_copy(bufb, t_hbm.at[k1 + 1, w])
        return 0
    lax.fori_loop(0, _K1 // 2, kpair, 0)


def _combine_body(coef_ref, t_ref, o_ref):
    o_ref[...] = jnp.maximum(
        lax.dot_general(coef_ref[...], t_ref[...], (((1,), (0,)), ((), ())),
                        preferred_element_type=jnp.float32), 0.0)


_CBLK = 12800


def _combine(tstack):
    tflat = tstack.reshape(_K1, _NW * _ROW)
    ncols = _NW * _ROW
    return pl.pallas_call(
        _combine_body,
        grid=(ncols // _CBLK,),
        in_specs=[
            pl.BlockSpec((_T, _K1), lambda i: (0, 0)),
            pl.BlockSpec((_K1, _CBLK), lambda i: (0, i)),
        ],
        out_specs=pl.BlockSpec((_T, _CBLK), lambda i: (0, i)),
        out_shape=jax.ShapeDtypeStruct((_T, ncols), jnp.float32),
    )(_COEFS, tflat)


def _lin_body(h_ref, w_ref, b_ref, o_ref):
    o_ref[...] = jnp.maximum(
        lax.dot_general(h_ref[...], w_ref[...], (((1,), (1,)), ((), ())),
                        preferred_element_type=jnp.float32) + b_ref[...], 0.0)


_LBLK = 2000


def _lin_relu(h, w, b):
    return pl.pallas_call(
        _lin_body,
        grid=(_N // _LBLK,),
        in_specs=[
            pl.BlockSpec((_LBLK, _T * _DIN), lambda i: (i, 0)),
            pl.BlockSpec(w.shape, lambda i: (0, 0)),
            pl.BlockSpec((1, w.shape[0]), lambda i: (0, 0)),
        ],
        out_specs=pl.BlockSpec((_LBLK, w.shape[0]), lambda i: (i, 0)),
        out_shape=jax.ShapeDtypeStruct((_N, w.shape[0]), jnp.float32),
    )(h, w, b.reshape(1, -1))


def _pool_body(b_ref, h_ref, s_ref, c_ref):
    i = pl.program_id(0)
    bb = b_ref[...].reshape(1, _LBLK)
    ids = lax.broadcasted_iota(jnp.int32, (_NG, _LBLK), 0)
    m = (ids == bb).astype(jnp.float32)
    ps = lax.dot_general(m, h_ref[...], (((1,), (0,)), ((), ())),
                         preferred_element_type=jnp.float32)
    pc = jnp.broadcast_to(jnp.sum(m, axis=1, keepdims=True), (_NG, 128))

    @pl.when(i == 0)
    def _():
        s_ref[...] = jnp.zeros_like(s_ref)
        c_ref[...] = jnp.zeros_like(c_ref)
    s_ref[...] += ps
    c_ref[...] += pc


def _pool(h, batch):
    b3 = batch.reshape(_N // _LBLK, 1, _LBLK)
    return pl.pallas_call(
        _pool_body,
        grid=(_N // _LBLK,),
        in_specs=[
            pl.BlockSpec((1, 1, _LBLK), lambda i: (i, 0, 0)),
            pl.BlockSpec((_LBLK, _T * _HID), lambda i: (i, 0)),
        ],
        out_specs=[
            pl.BlockSpec((_NG, _T * _HID), lambda i: (0, 0)),
            pl.BlockSpec((_NG, 128), lambda i: (0, 0)),
        ],
        out_shape=[
            jax.ShapeDtypeStruct((_NG, _T * _HID), jnp.float32),
            jax.ShapeDtypeStruct((_NG, 128), jnp.float32),
        ],
    )(b3, h)


def _head_body(s_ref, c_ref, w2_ref, b2_ref, w3_ref, b3_ref, w4_ref, b4_ref,
               o_ref):
    cnt = jnp.maximum(c_ref[:, 0:1], 1.0)
    g = s_ref[...] / cnt
    g = lax.dot_general(g, w2_ref[...], (((1,), (1,)), ((), ())),
                        preferred_element_type=jnp.float32) + b2_ref[...]
    g = jnp.maximum(g, 0.0)
    g = lax.dot_general(g, w3_ref[...], (((1,), (1,)), ((), ())),
                        preferred_element_type=jnp.float32) + b3_ref[...]
    g = jnp.maximum(g, 0.0)
    g = lax.dot_general(g, w4_ref[...], (((1,), (1,)), ((), ())),
                        preferred_element_type=jnp.float32) + b4_ref[...]
    mx = jnp.max(g, axis=1, keepdims=True)
    lse = mx + jnp.log(jnp.sum(jnp.exp(g - mx), axis=1, keepdims=True))
    o_ref[...] = g - lse


def _head(sums, cnts, lin2_W, lin2_b, cls1_W, cls1_b, cls2_W, cls2_b):
    args = (sums, cnts, lin2_W, lin2_b.reshape(1, -1), cls1_W,
            cls1_b.reshape(1, -1), cls2_W, cls2_b.reshape(1, -1))
    return pl.pallas_call(
        _head_body,
        out_shape=jax.ShapeDtypeStruct((_NG, _NOUT), jnp.float32),
    )(*args)


def _to_tiles(h):
    return h.reshape(_N, _NW, _FPW).transpose(1, 0, 2).reshape(_NW, _ROW)


def _from_combine(c):
    return c.reshape(_T, _NW, _N, _FPW).transpose(2, 0, 1, 3).reshape(_N, -1)


def kernel(x, edge_index, edge_weight, batch, lin1_W, lin1_b, lin2_W, lin2_b,
           cls1_W, cls1_b, cls2_W, cls2_b):
    src = edge_index[0].astype(jnp.int32)
    dst = edge_index[1].astype(jnp.int32)
    packed = src | (dst << 16)
    norm = _norm_kernel(packed, edge_weight)

    t1 = _layer_kernel(_to_tiles(x), packed, norm)
    h = _lin_relu(_from_combine(_combine(t1)), lin1_W, lin1_b)

    t2 = _layer_kernel(_to_tiles(h), packed, norm)
    h2 = _from_combine(_combine(t2))

    sums, cnts = _pool(h2, batch)
    return _head(sums, cnts, lin2_W, lin2_b, cls1_W, cls1_b, cls2_W, cls2_b)


# speedup vs baseline: 4.7867x; 1.4259x over previous
import functools

import jax
import jax.numpy as jnp
import numpy as np
from jax import lax
from jax.experimental import pallas as pl
from jax.experimental.pallas import tpu as pltpu
from jax.experimental.pallas import tpu_sc as plsc

_TS = np.array([0.25, 0.5, 0.75, 1.0], dtype=np.float64)
_DEGREE = 32
_N = 10000
_E = 320000
_DIN = 128
_HID = 128
_NOUT = 10
_NG = 64
_T = len(_TS)

_NW = 32
_FPW = _DIN // _NW
_ROW = _N * _FPW
_CH = 8000
_NCH = _E // _CH
_NPAIR = _NCH // 2
_NGRP = _CH // 16
_SLICE = _E // _NW
_PCH = 2000
_K1 = _DEGREE + 1


def _cheb_coefs():
    nint = 1000
    theta = np.linspace(0.0, np.pi, nint)
    dtheta = theta[1] - theta[0]
    lam = np.cos(theta) + 1.0
    coefs = np.zeros((_T, _K1), dtype=np.float64)
    for i, t in enumerate(_TS):
        f = np.exp(-t * lam)
        for k in range(_K1):
            y = f * np.cos(k * theta)
            a = (2.0 / np.pi) * dtheta * (0.5 * y[0] + 0.5 * y[-1] + np.sum(y[1:-1]))
            if k == 0:
                a *= 0.5
            coefs[i, k] = a
    return jnp.asarray(coefs, dtype=jnp.float32)


_COEFS = _cheb_coefs()

_sc_mesh = plsc.VectorSubcoreMesh(core_axis_name="c", subcore_axis_name="s")
_sc_params = pltpu.CompilerParams(needs_layout_passes=False)


def _worker_id():
    return lax.axis_index("s") * 2 + lax.axis_index("c")


def _unpack_src(pk):
    return pk & jnp.int32(0xFFFF)


def _unpack_dst(pk):
    return pk >> 16


@functools.partial(
    pl.kernel,
    out_type=jax.ShapeDtypeStruct((_E,), jnp.float32),
    mesh=_sc_mesh,
    compiler_params=_sc_params,
    scratch_types=[
        pltpu.VMEM((_N,), jnp.float32),
        pltpu.VMEM((_PCH,), jnp.int32),
        pltpu.VMEM((_PCH,), jnp.float32),
        pltpu.VMEM((_PCH,), jnp.float32),
    ],
)
def _norm_kernel(pk_hbm, w_hbm, nm_hbm, deg, pkb, wb, nb):
    w = _worker_id()
    zero16 = jnp.zeros((16,), jnp.float32)

    def z_body(i, _):
        deg[pl.ds(i * 16, 16)] = zero16
        return 0
    lax.fori_loop(0, _N // 16, z_body, 0)

    def deg_chunk(ci, _):
        pltpu.sync_copy(pk_hbm.at[pl.ds(ci * _PCH, _PCH)], pkb)
        pltpu.sync_copy(w_hbm.at[pl.ds(ci * _PCH, _PCH)], wb)

        def grp(g, _):
            pk = pkb[pl.ds(g * 16, 16)]
            ww = wb[pl.ds(g * 16, 16)]
            plsc.addupdate_scatter(deg, [pk >> 16], ww)
            return 0
        lax.fori_loop(0, _PCH // 16, grp, 0)
        return 0
    lax.fori_loop(0, _E // _PCH, deg_chunk, 0)

    def rs_body(i, _):
        d = deg[pl.ds(i * 16, 16)]
        bits = plsc.bitcast(d, jnp.int32)
        y = plsc.bitcast(jnp.int32(0x5F3759DF) - (bits >> 1), jnp.float32)
        for _i in range(3):
            y = y * (1.5 - 0.5 * d * y * y)
        deg[pl.ds(i * 16, 16)] = jnp.where(d > 0.0, y, 0.0)
        return 0
    lax.fori_loop(0, _N // 16, rs_body, 0)

    base = w * _SLICE

    def nm_chunk(ci, _):
        off = base + ci * _PCH
        pltpu.sync_copy(pk_hbm.at[pl.ds(off, _PCH)], pkb)
        pltpu.sync_copy(w_hbm.at[pl.ds(off, _PCH)], wb)

        def grp(g, _):
            pk = pkb[pl.ds(g * 16, 16)]
            ww = wb[pl.ds(g * 16, 16)]
            gs = plsc.load_gather(deg, [pk & jnp.int32(0xFFFF)])
            gd = plsc.load_gather(deg, [pk >> 16])
            nb[pl.ds(g * 16, 16)] = gs * ww * gd
            return 0
        lax.fori_loop(0, _PCH // 16, grp, 0)
        pltpu.sync_copy(nb, nm_hbm.at[pl.ds(off, _PCH)])
        return 0
    lax.fori_loop(0, _SLICE // _PCH, nm_chunk, 0)


@functools.partial(
    pl.kernel,
    out_type=jax.ShapeDtypeStruct((_K1, _NW, _ROW), jnp.float32),
    mesh=_sc_mesh,
    compiler_params=_sc_params,
    scratch_types=[
        pltpu.VMEM((_ROW,), jnp.float32),
        pltpu.VMEM((_ROW,), jnp.float32),
        pltpu.VMEM((_CH,), jnp.int32),
        pltpu.VMEM((_CH,), jnp.int32),
        pltpu.VMEM((_CH,), jnp.float32),
        pltpu.VMEM((_CH,), jnp.float32),
        pltpu.SemaphoreType.DMA,
        pltpu.SemaphoreType.DMA,
        pltpu.SemaphoreType.DMA,
        pltpu.SemaphoreType.DMA,
    ],
)
def _layer_kernel(xt_hbm, pk_hbm, nm_hbm, t_hbm, bufa, bufb, pk0, pk1,
                  nm0, nm1, sp0, sp1, sn0, sn1):
    w = _worker_id()
    zero16 = jnp.zeros((16,), jnp.float32)

    def start(ci, pkb, nmb, sp, sn):
        pltpu.make_async_copy(pk_hbm.at[pl.ds(ci * _CH, _CH)], pkb, sp).start()
        pltpu.make_async_copy(nm_hbm.at[pl.ds(ci * _CH, _CH)], nmb, sn).start()

    def wait(pkb, nmb, sp, sn):
        pltpu.make_async_copy(pk_hbm.at[pl.ds(0, _CH)], pkb, sp).wait()
        pltpu.make_async_copy(nm_hbm.at[pl.ds(0, _CH)], nmb, sn).wait()

    def process(pkb, nmb, srcbuf, accbuf, factor):
        @plsc.parallel_loop(0, _NGRP, unroll=8)
        def grp(g):
            pk = pkb[pl.ds(g * 16, 16)]
            nm = nmb[pl.ds(g * 16, 16)]
            sb = _unpack_src(pk)
            db = _unpack_dst(pk)
            nf = nm * factor
            for f in range(_FPW):
                gf = plsc.load_gather(srcbuf, [sb + f * _N])
                plsc.addupdate_scatter(accbuf, [db + f * _N], gf * nf)

    def edge_pass(srcbuf, accbuf, factor):
        start(0, pk0, nm0, sp0, sn0)
        start(1, pk1, nm1, sp1, sn1)

        def pair(p, _):
            wait(pk0, nm0, sp0, sn0)
            process(pk0, nm0, srcbuf, accbuf, factor)

            @pl.when(p < _NPAIR - 1)
            def _():
                start(2 * p + 2, pk0, nm0, sp0, sn0)
            wait(pk1, nm1, sp1, sn1)
            process(pk1, nm1, srcbuf, accbuf, factor)

            @pl.when(p < _NPAIR - 1)
            def _():
                start(2 * p + 3, pk1, nm1, sp1, sn1)
            return 0
        lax.fori_loop(0, _NPAIR, pair, 0)

    def negate(buf):
        @plsc.parallel_loop(0, _ROW // 16, unroll=8)
        def it(i):
            buf[pl.ds(i * 16, 16)] = -buf[pl.ds(i * 16, 16)]

    pltpu.sync_copy(xt_hbm.at[w], bufb)
    pltpu.sync_copy(bufb, t_hbm.at[0, w])

    @plsc.parallel_loop(0, _ROW // 16, unroll=8)
    def z_body(i):
        bufa[pl.ds(i * 16, 16)] = zero16

    def kpair(kk, _):
        k1 = 2 * kk + 1

        @pl.when(k1 > 1)
        def _():
            negate(bufa)
        f1 = jnp.where(k1 == 1, jnp.float32(-1.0), jnp.float32(-2.0))
        edge_pass(bufb, bufa, f1)
        pltpu.sync_copy(bufa, t_hbm.at[k1, w])
        negate(bufb)
        edge_pass(bufa, bufb, jnp.float32(-2.0))
        pltpu.sync_copy(bufb, t_hbm.at[k1 + 1, w])
        return 0
    lax.fori_loop(0, _K1 // 2, kpair, 0)


def _combine_body(coef_ref, t_ref, o_ref):
    o_ref[...] = jnp.maximum(
        lax.dot_general(coef_ref[...], t_ref[...], (((1,), (0,)), ((), ())),
                        preferred_element_type=jnp.float32), 0.0)


_CBLK = 12800


def _combine(tstack):
    tflat = tstack.reshape(_K1, _NW * _ROW)
    ncols = _NW * _ROW
    return pl.pallas_call(
        _combine_body,
        grid=(ncols // _CBLK,),
        in_specs=[
            pl.BlockSpec((_T, _K1), lambda i: (0, 0)),
            pl.BlockSpec((_K1, _CBLK), lambda i: (0, i)),
        ],
        out_specs=pl.BlockSpec((_T, _CBLK), lambda i: (0, i)),
        out_shape=jax.ShapeDtypeStruct((_T, ncols), jnp.float32),
    )(_COEFS, tflat)


def _lin_body(h_ref, w_ref, b_ref, o_ref):
    o_ref[...] = jnp.maximum(
        lax.dot_general(h_ref[...], w_ref[...], (((1,), (1,)), ((), ())),
                        preferred_element_type=jnp.float32) + b_ref[...], 0.0)


_LBLK = 2000


def _lin_relu(h, w, b):
    return pl.pallas_call(
        _lin_body,
        grid=(_N // _LBLK,),
        in_specs=[
            pl.BlockSpec((_LBLK, _T * _DIN), lambda i: (i, 0)),
            pl.BlockSpec(w.shape, lambda i: (0, 0)),
            pl.BlockSpec((1, w.shape[0]), lambda i: (0, 0)),
        ],
        out_specs=pl.BlockSpec((_LBLK, w.shape[0]), lambda i: (i, 0)),
        out_shape=jax.ShapeDtypeStruct((_N, w.shape[0]), jnp.float32),
    )(h, w, b.reshape(1, -1))


def _pool_body(b_ref, h_ref, s_ref, c_ref):
    i = pl.program_id(0)
    bb = b_ref[...].reshape(1, _LBLK)
    ids = lax.broadcasted_iota(jnp.int32, (_NG, _LBLK), 0)
    m = (ids == bb).astype(jnp.float32)
    ps = lax.dot_general(m, h_ref[...], (((1,), (0,)), ((), ())),
                         preferred_element_type=jnp.float32)
    pc = jnp.broadcast_to(jnp.sum(m, axis=1, keepdims=True), (_NG, 128))

    @pl.when(i == 0)
    def _():
        s_ref[...] = jnp.zeros_like(s_ref)
        c_ref[...] = jnp.zeros_like(c_ref)
    s_ref[...] += ps
    c_ref[...] += pc


def _pool(h, batch):
    b3 = batch.reshape(_N // _LBLK, 1, _LBLK)
    return pl.pallas_call(
        _pool_body,
        grid=(_N // _LBLK,),
        in_specs=[
            pl.BlockSpec((1, 1, _LBLK), lambda i: (i, 0, 0)),
            pl.BlockSpec((_LBLK, _T * _HID), lambda i: (i, 0)),
        ],
        out_specs=[
            pl.BlockSpec((_NG, _T * _HID), lambda i: (0, 0)),
            pl.BlockSpec((_NG, 128), lambda i: (0, 0)),
        ],
        out_shape=[
            jax.ShapeDtypeStruct((_NG, _T * _HID), jnp.float32),
            jax.ShapeDtypeStruct((_NG, 128), jnp.float32),
        ],
    )(b3, h)


def _head_body(s_ref, c_ref, w2_ref, b2_ref, w3_ref, b3_ref, w4_ref, b4_ref,
               o_ref):
    cnt = jnp.maximum(c_ref[:, 0:1], 1.0)
    g = s_ref[...] / cnt
    g = lax.dot_general(g, w2_ref[...], (((1,), (1,)), ((), ())),
                        preferred_element_type=jnp.float32) + b2_ref[...]
    g = jnp.maximum(g, 0.0)
    g = lax.dot_general(g, w3_ref[...], (((1,), (1,)), ((), ())),
                        preferred_element_type=jnp.float32) + b3_ref[...]
    g = jnp.maximum(g, 0.0)
    g = lax.dot_general(g, w4_ref[...], (((1,), (1,)), ((), ())),
                        preferred_element_type=jnp.float32) + b4_ref[...]
    mx = jnp.max(g, axis=1, keepdims=True)
    lse = mx + jnp.log(jnp.sum(jnp.exp(g - mx), axis=1, keepdims=True))
    o_ref[...] = g - lse


def _head(sums, cnts, lin2_W, lin2_b, cls1_W, cls1_b, cls2_W, cls2_b):
    args = (sums, cnts, lin2_W, lin2_b.reshape(1, -1), cls1_W,
            cls1_b.reshape(1, -1), cls2_W, cls2_b.reshape(1, -1))
    return pl.pallas_call(
        _head_body,
        out_shape=jax.ShapeDtypeStruct((_NG, _NOUT), jnp.float32),
    )(*args)


def _to_tiles(h):
    return h.reshape(_N, _NW, _FPW).transpose(1, 2, 0).reshape(_NW, _ROW)


def _from_combine(c):
    return c.reshape(_T, _NW, _FPW, _N).transpose(3, 0, 1, 2).reshape(_N, -1)


def kernel(x, edge_index, edge_weight, batch, lin1_W, lin1_b, lin2_W, lin2_b,
           cls1_W, cls1_b, cls2_W, cls2_b):
    src = edge_index[0].astype(jnp.int32)
    dst = edge_index[1].astype(jnp.int32)
    packed = src | (dst << 16)
    norm = _norm_kernel(packed, edge_weight)

    t1 = _layer_kernel(_to_tiles(x), packed, norm)
    h = _lin_relu(_from_combine(_combine(t1)), lin1_W, lin1_b)

    t2 = _layer_kernel(_to_tiles(h), packed, norm)
    h2 = _from_combine(_combine(t2))

    sums, cnts = _pool(h2, batch)
    return _head(sums, cnts, lin2_W, lin2_b, cls1_W, cls1_b, cls2_W, cls2_b)


# speedup vs baseline: 7.5573x; 1.5788x over previous
import functools

import jax
import jax.numpy as jnp
import numpy as np
from jax import lax
from jax.experimental import pallas as pl
from jax.experimental.pallas import tpu as pltpu
from jax.experimental.pallas import tpu_sc as plsc

_TS = np.array([0.25, 0.5, 0.75, 1.0], dtype=np.float64)
_DEGREE = 32
_N = 10000
_E = 320000
_DIN = 128
_HID = 128
_NOUT = 10
_NG = 64
_T = len(_TS)

_NW = 32
_FPW = _DIN // _NW
_ROW = _N * _FPW
_CH = 8000
_NCH = _E // _CH
_NPAIR = _NCH // 2
_NGRP = _CH // 16
_SLICE = _E // _NW
_PCH = 2000
_K1 = _DEGREE + 1


def _cheb_coefs():
    nint = 1000
    theta = np.linspace(0.0, np.pi, nint)
    dtheta = theta[1] - theta[0]
    lam = np.cos(theta) + 1.0
    coefs = np.zeros((_T, _K1), dtype=np.float64)
    for i, t in enumerate(_TS):
        f = np.exp(-t * lam)
        for k in range(_K1):
            y = f * np.cos(k * theta)
            a = (2.0 / np.pi) * dtheta * (0.5 * y[0] + 0.5 * y[-1] + np.sum(y[1:-1]))
            if k == 0:
                a *= 0.5
            coefs[i, k] = a
    return jnp.asarray(coefs, dtype=jnp.float32)


_COEFS_NP = np.asarray(_cheb_coefs())
_COEFS = jnp.asarray(_COEFS_NP)

_sc_mesh = plsc.VectorSubcoreMesh(core_axis_name="c", subcore_axis_name="s")
_sc_params = pltpu.CompilerParams(needs_layout_passes=False)


def _worker_id():
    return lax.axis_index("s") * 2 + lax.axis_index("c")


def _unpack_src(pk):
    return pk & jnp.int32(0xFFFF)


def _unpack_dst(pk):
    return pk >> 16


@functools.partial(
    pl.kernel,
    out_type=jax.ShapeDtypeStruct((_E,), jnp.float32),
    mesh=_sc_mesh,
    compiler_params=_sc_params,
    scratch_types=[
        pltpu.VMEM((_N,), jnp.float32),
        pltpu.VMEM((_PCH,), jnp.int32),
        pltpu.VMEM((_PCH,), jnp.float32),
        pltpu.VMEM((_PCH,), jnp.float32),
    ],
)
def _norm_kernel(pk_hbm, w_hbm, nm_hbm, deg, pkb, wb, nb):
    w = _worker_id()
    zero16 = jnp.zeros((16,), jnp.float32)

    def z_body(i, _):
        deg[pl.ds(i * 16, 16)] = zero16
        return 0
    lax.fori_loop(0, _N // 16, z_body, 0)

    def deg_chunk(ci, _):
        pltpu.sync_copy(pk_hbm.at[pl.ds(ci * _PCH, _PCH)], pkb)
        pltpu.sync_copy(w_hbm.at[pl.ds(ci * _PCH, _PCH)], wb)

        def grp(g, _):
            pk = pkb[pl.ds(g * 16, 16)]
            ww = wb[pl.ds(g * 16, 16)]
            plsc.addupdate_scatter(deg, [pk >> 16], ww)
            return 0
        lax.fori_loop(0, _PCH // 16, grp, 0)
        return 0
    lax.fori_loop(0, _E // _PCH, deg_chunk, 0)

    def rs_body(i, _):
        d = deg[pl.ds(i * 16, 16)]
        bits = plsc.bitcast(d, jnp.int32)
        y = plsc.bitcast(jnp.int32(0x5F3759DF) - (bits >> 1), jnp.float32)
        for _i in range(3):
            y = y * (1.5 - 0.5 * d * y * y)
        deg[pl.ds(i * 16, 16)] = jnp.where(d > 0.0, y, 0.0)
        return 0
    lax.fori_loop(0, _N // 16, rs_body, 0)

    base = w * _SLICE

    def nm_chunk(ci, _):
        off = base + ci * _PCH
        pltpu.sync_copy(pk_hbm.at[pl.ds(off, _PCH)], pkb)
        pltpu.sync_copy(w_hbm.at[pl.ds(off, _PCH)], wb)

        def grp(g, _):
            pk = pkb[pl.ds(g * 16, 16)]
            ww = wb[pl.ds(g * 16, 16)]
            gs = plsc.load_gather(deg, [pk & jnp.int32(0xFFFF)])
            gd = plsc.load_gather(deg, [pk >> 16])
            nb[pl.ds(g * 16, 16)] = gs * ww * gd
            return 0
        lax.fori_loop(0, _PCH // 16, grp, 0)
        pltpu.sync_copy(nb, nm_hbm.at[pl.ds(off, _PCH)])
        return 0
    lax.fori_loop(0, _SLICE // _PCH, nm_chunk, 0)


@functools.partial(
    pl.kernel,
    out_type=jax.ShapeDtypeStruct((_K1, _NW, _ROW), jnp.float32),
    mesh=_sc_mesh,
    compiler_params=_sc_params,
    scratch_types=[
        pltpu.VMEM((_ROW,), jnp.float32),
        pltpu.VMEM((_ROW,), jnp.float32),
        pltpu.VMEM((_CH,), jnp.int32),
        pltpu.VMEM((_CH,), jnp.int32),
        pltpu.VMEM((_CH,), jnp.float32),
        pltpu.VMEM((_CH,), jnp.float32),
        pltpu.SemaphoreType.DMA,
        pltpu.SemaphoreType.DMA,
        pltpu.SemaphoreType.DMA,
        pltpu.SemaphoreType.DMA,
    ],
)
def _layer_kernel(xt_hbm, pk_hbm, nm_hbm, t_hbm, bufa, bufb, pk0, pk1,
                  nm0, nm1, sp0, sp1, sn0, sn1):
    w = _worker_id()
    zero16 = jnp.zeros((16,), jnp.float32)

    def start(ci, pkb, nmb, sp, sn):
        pltpu.make_async_copy(pk_hbm.at[pl.ds(ci * _CH, _CH)], pkb, sp).start()
        pltpu.make_async_copy(nm_hbm.at[pl.ds(ci * _CH, _CH)], nmb, sn).start()

    def wait(pkb, nmb, sp, sn):
        pltpu.make_async_copy(pk_hbm.at[pl.ds(0, _CH)], pkb, sp).wait()
        pltpu.make_async_copy(nm_hbm.at[pl.ds(0, _CH)], nmb, sn).wait()

    def process(pkb, nmb, srcbuf, accbuf, factor):
        @plsc.parallel_loop(0, _NGRP, unroll=8)
        def grp(g):
            pk = pkb[pl.ds(g * 16, 16)]
            nm = nmb[pl.ds(g * 16, 16)]
            sb = _unpack_src(pk)
            db = _unpack_dst(pk)
            nf = nm * factor
            for f in range(_FPW):
                gf = plsc.load_gather(srcbuf, [sb + f * _N])
                plsc.addupdate_scatter(accbuf, [db + f * _N], gf * nf)

    def edge_pass(srcbuf, accbuf, factor):
        start(0, pk0, nm0, sp0, sn0)
        start(1, pk1, nm1, sp1, sn1)

        def pair(p, _):
            wait(pk0, nm0, sp0, sn0)
            process(pk0, nm0, srcbuf, accbuf, factor)

            @pl.when(p < _NPAIR - 1)
            def _():
                start(2 * p + 2, pk0, nm0, sp0, sn0)
            wait(pk1, nm1, sp1, sn1)
            process(pk1, nm1, srcbuf, accbuf, factor)

            @pl.when(p < _NPAIR - 1)
            def _():
                start(2 * p + 3, pk1, nm1, sp1, sn1)
            return 0
        lax.fori_loop(0, _NPAIR, pair, 0)

    def negate(buf):
        @plsc.parallel_loop(0, _ROW // 16, unroll=8)
        def it(i):
            buf[pl.ds(i * 16, 16)] = -buf[pl.ds(i * 16, 16)]

    pltpu.sync_copy(xt_hbm.at[w], bufb)
    pltpu.sync_copy(bufb, t_hbm.at[0, w])

    @plsc.parallel_loop(0, _ROW // 16, unroll=8)
    def z_body(i):
        bufa[pl.ds(i * 16, 16)] = zero16

    def kpair(kk, _):
        k1 = 2 * kk + 1

        @pl.when(k1 > 1)
        def _():
            negate(bufa)
        f1 = jnp.where(k1 == 1, jnp.float32(-1.0), jnp.float32(-2.0))
        edge_pass(bufb, bufa, f1)
        pltpu.sync_copy(bufa, t_hbm.at[k1, w])
        negate(bufb)
        edge_pass(bufa, bufb, jnp.float32(-2.0))
        pltpu.sync_copy(bufb, t_hbm.at[k1 + 1, w])
        return 0
    lax.fori_loop(0, _K1 // 2, kpair, 0)


_WFB = 8


def _combine_t(t_ref, t):
    acc = t_ref[0] * jnp.float32(_COEFS_NP[t, 0])
    for k in range(1, _K1):
        acc = acc + t_ref[k] * jnp.float32(_COEFS_NP[t, k])
    return jnp.maximum(acc, 0.0)


def _comb_lin1_body(t_ref, w_ref, b_ref, o_ref):
    i = pl.program_id(0)
    part = None
    for t in range(_T):
        cb = _combine_t(t_ref, t)
        pt = lax.dot_general(w_ref[0, :, t, :], cb,
                             (((1,), (0,)), ((), ())),
                             preferred_element_type=jnp.float32)
        part = pt if part is None else part + pt

    @pl.when(i == 0)
    def _():
        o_ref[...] = jnp.zeros_like(o_ref)
    o_ref[...] += part

    @pl.when(i == _DIN // _WFB - 1)
    def _():
        o_ref[...] = jnp.maximum(o_ref[...] + b_ref[...], 0.0)


def _comb_lin1(tstack, w, b):
    tv = tstack.reshape(_K1, _DIN, _N)
    w4 = (w.reshape(_HID, _T, _DIN // _WFB, _WFB)
          .transpose(2, 0, 1, 3))
    return pl.pallas_call(
        _comb_lin1_body,
        grid=(_DIN // _WFB,),
        in_specs=[
            pl.BlockSpec((_K1, _WFB, _N), lambda i: (0, i, 0)),
            pl.BlockSpec((1, _HID, _T, _WFB), lambda i: (i, 0, 0, 0)),
            pl.BlockSpec((_HID, 1), lambda i: (0, 0)),
        ],
        out_specs=pl.BlockSpec((_HID, _N), lambda i: (0, 0)),
        out_shape=jax.ShapeDtypeStruct((_HID, _N), jnp.float32),
    )(tv, w4, b.reshape(_HID, 1))


def _comb_pool_body(t_ref, b_ref, s_ref, c_ref):
    i = pl.program_id(0)
    ids = lax.broadcasted_iota(jnp.int32, (_N, _NG), 1)
    m = (ids == b_ref[...]).astype(jnp.float32)
    for t in range(_T):
        cb = _combine_t(t_ref, t)
        s_ref[pl.ds(t * _WFB, _WFB), :] = lax.dot_general(
            cb, m, (((1,), (0,)), ((), ())),
            preferred_element_type=jnp.float32)

    @pl.when(i == 0)
    def _():
        c_ref[...] = jnp.broadcast_to(jnp.sum(m, axis=0, keepdims=True),
                                      (8, _NG))


def _comb_pool(tstack, batch):
    tv = tstack.reshape(_K1, _DIN, _N)
    return pl.pallas_call(
        _comb_pool_body,
        grid=(_DIN // _WFB,),
        in_specs=[
            pl.BlockSpec((_K1, _WFB, _N), lambda i: (0, i, 0)),
            pl.BlockSpec((_N, 1), lambda i: (0, 0)),
        ],
        out_specs=[
            pl.BlockSpec((_T * _WFB, _NG), lambda i: (i, 0)),
            pl.BlockSpec((8, _NG), lambda i: (0, 0)),
        ],
        out_shape=[
            jax.ShapeDtypeStruct((_T * _DIN, _NG), jnp.float32),
            jax.ShapeDtypeStruct((8, _NG), jnp.float32),
        ],
    )(tv, batch.reshape(_N, 1))


def _head_body(s_ref, c_ref, w2_ref, b2_ref, w3_ref, b3_ref, w4_ref, b4_ref,
               o_ref):
    cnt = jnp.maximum(c_ref[0:1, :], 1.0)
    g = s_ref[...] / cnt
    g = lax.dot_general(w2_ref[...], g, (((1,), (0,)), ((), ())),
                        preferred_element_type=jnp.float32) + b2_ref[...]
    g = jnp.maximum(g, 0.0)
    g = lax.dot_general(w3_ref[...], g, (((1,), (0,)), ((), ())),
                        preferred_element_type=jnp.float32) + b3_ref[...]
    g = jnp.maximum(g, 0.0)
    g = lax.dot_general(w4_ref[...], g, (((1,), (0,)), ((), ())),
                        preferred_element_type=jnp.float32) + b4_ref[...]
    mx = jnp.max(g, axis=0, keepdims=True)
    lse = mx + jnp.log(jnp.sum(jnp.exp(g - mx), axis=0, keepdims=True))
    o_ref[...] = g - lse


def _head(sums, cnts, lin2_W, lin2_b, cls1_W, cls1_b, cls2_W, cls2_b):
    args = (sums, cnts, lin2_W, lin2_b.reshape(-1, 1), cls1_W,
            cls1_b.reshape(-1, 1), cls2_W, cls2_b.reshape(-1, 1))
    return pl.pallas_call(
        _head_body,
        out_shape=jax.ShapeDtypeStruct((_NOUT, _NG), jnp.float32),
    )(*args)


def _to_tiles(h):
    return h.reshape(_N, _NW, _FPW).transpose(1, 2, 0).reshape(_NW, _ROW)


def kernel(x, edge_index, edge_weight, batch, lin1_W, lin1_b, lin2_W, lin2_b,
           cls1_W, cls1_b, cls2_W, cls2_b):
    src = edge_index[0].astype(jnp.int32)
    dst = edge_index[1].astype(jnp.int32)
    packed = src | (dst << 16)
    norm = _norm_kernel(packed, edge_weight)

    t1 = _layer_kernel(_to_tiles(x), packed, norm)
    h = _comb_lin1(t1, lin1_W, lin1_b)

    t2 = _layer_kernel(h.reshape(_NW, _ROW), packed, norm)
    sums, cnts = _comb_pool(t2, batch)
    lin2p = (lin2_W.reshape(_HID, _T, _DIN // _WFB, _WFB)
             .transpose(0, 2, 1, 3).reshape(_HID, _T * _DIN))
    out_t = _head(sums, cnts, lin2p, lin2_b, cls1_W, cls1_b, cls2_W, cls2_b)
    return out_t.T
